# Initial kernel scaffold; baseline (speedup 1.0000x reference)
import jax
import jax.numpy as jnp
from jax.experimental import pallas as pl


def kernel(f_atoms, f_bonds, a2b, b2a, b2revb, a_scope, W_i_atom, W_i_bond, W_h_0, W_h_1, lr_W, W_o_W, W_o_b, gru_bias, W_ih_f, W_hh_f, b_ih_f, b_hh_f, W_ih_r, W_hh_r, b_ih_r, b_hh_r):
    raise NotImplementedError("write your pallas kernel here")


# speedup vs baseline: 1.8162x; 1.8162x over previous
import functools

import jax
import jax.numpy as jnp
from jax import lax
from jax.experimental import pallas as pl
from jax.experimental.pallas import tpu as pltpu
from jax.experimental.pallas import tpu_sc as plsc

H = 128
N_MOL = 1024
APM = 48
N_ATOMS = 1 + N_MOL * APM
N_BONDS = 1 + N_MOL * APM * 4
MAX_NB = 6

A_PAD = 49664
B_PAD = 198656
NEI = A_PAD * MAX_NB

_NW = 32


def _sc_gather(table, idx, chunk):
    b = idx.shape[0]
    bpw = b // _NW
    nsteps = bpw // chunk
    mesh = plsc.VectorSubcoreMesh(core_axis_name="c", subcore_axis_name="s")

    @functools.partial(
        pl.kernel,
        mesh=mesh,
        out_type=jax.ShapeDtypeStruct((b, H), jnp.float32),
        scratch_types=[
            pltpu.VMEM((chunk,), jnp.int32),
            pltpu.VMEM((chunk, H), jnp.float32),
            pltpu.SemaphoreType.DMA,
        ],
    )
    def k(table_hbm, idx_hbm, out_hbm, idx_v, rows_v, sem):
        wid = lax.axis_index("s") * 2 + lax.axis_index("c")
        base = wid * bpw

        def step(g, carry):
            off = base + g * chunk
            pltpu.sync_copy(idx_hbm.at[pl.ds(off, chunk)], idx_v)
            pltpu.async_copy(table_hbm.at[idx_v], rows_v, sem).wait()
            pltpu.sync_copy(rows_v, out_hbm.at[pl.ds(off, chunk)])
            return carry

        lax.fori_loop(0, nsteps, step, 0)

    return k(table, idx)


def _mm_relu_body(x_ref, wt_ref, o_ref):
    o_ref[...] = jax.nn.relu(
        jnp.dot(x_ref[...], wt_ref[...], preferred_element_type=jnp.float32))


def _mm_relu(x, wt, bn):
    n, k = x.shape
    return pl.pallas_call(
        _mm_relu_body,
        grid=(n // bn,),
        in_specs=[pl.BlockSpec((bn, k), lambda i: (i, 0)),
                  pl.BlockSpec((k, H), lambda i: (0, 0))],
        out_specs=pl.BlockSpec((bn, H), lambda i: (i, 0)),
        out_shape=jax.ShapeDtypeStruct((n, H), jnp.float32),
    )(x, wt)


def _agg_base_body(nei_ref, base_ref, o_ref):
    x = nei_ref[...]
    o_ref[...] = base_ref[...] + x.sum(axis=1) * x.max(axis=1)


def _agg_nb_body(nei_ref, o_ref):
    x = nei_ref[...]
    o_ref[...] = x.sum(axis=1) * x.max(axis=1)


_BA = 128


def _agg_base(nei3, base):
    return pl.pallas_call(
        _agg_base_body,
        grid=(A_PAD // _BA,),
        in_specs=[pl.BlockSpec((_BA, MAX_NB, H), lambda i: (i, 0, 0)),
                  pl.BlockSpec((_BA, H), lambda i: (i, 0))],
        out_specs=pl.BlockSpec((_BA, H), lambda i: (i, 0)),
        out_shape=jax.ShapeDtypeStruct((A_PAD, H), jnp.float32),
    )(nei3, base)


def _agg_nb(nei3):
    return pl.pallas_call(
        _agg_nb_body,
        grid=(A_PAD // _BA,),
        in_specs=[pl.BlockSpec((_BA, MAX_NB, H), lambda i: (i, 0, 0))],
        out_specs=pl.BlockSpec((_BA, H), lambda i: (i, 0)),
        out_shape=jax.ShapeDtypeStruct((A_PAD, H), jnp.float32),
    )(nei3)


def _bond_body(ga_ref, gb_ref, ib_ref, wt_ref, o_ref):
    pre = ga_ref[...] - gb_ref[...]
    o_ref[...] = jax.nn.relu(
        ib_ref[...] +
        jnp.dot(pre, wt_ref[...], preferred_element_type=jnp.float32))


def _bond_update(ga, gb, ib, wh_t):
    bn = 512
    return pl.pallas_call(
        _bond_body,
        grid=(B_PAD // bn,),
        in_specs=[pl.BlockSpec((bn, H), lambda i: (i, 0)),
                  pl.BlockSpec((bn, H), lambda i: (i, 0)),
                  pl.BlockSpec((bn, H), lambda i: (i, 0)),
                  pl.BlockSpec((H, H), lambda i: (0, 0))],
        out_specs=pl.BlockSpec((bn, H), lambda i: (i, 0)),
        out_shape=jax.ShapeDtypeStruct((B_PAD, H), jnp.float32),
    )(ga, gb, ib, wh_t)


_BM5 = 8


def _node_body(agg_ref, ma_ref, ia_ref, l0_ref, l1_ref, l2_ref, gb_ref,
               wif_ref, bif_ref, wir_ref, bir_ref,
               gif_ref, gir_ref, h0_ref):
    node = (jnp.dot(agg_ref[...], l0_ref[...], preferred_element_type=jnp.float32)
            + jnp.dot(ma_ref[...], l1_ref[...], preferred_element_type=jnp.float32)
            + jnp.dot(ia_ref[...], l2_ref[...], preferred_element_type=jnp.float32))
    h0_ref[...] = node.reshape(_BM5, APM, H).max(axis=1)
    msg = jax.nn.relu(node + gb_ref[...])
    gif = jnp.dot(msg, wif_ref[...], preferred_element_type=jnp.float32) + bif_ref[...]
    gir = jnp.dot(msg, wir_ref[...], preferred_element_type=jnp.float32) + bir_ref[...]
    gif_ref[...] = gif.reshape(_BM5, APM, 3 * H).swapaxes(0, 1)
    gir_ref[...] = gir.reshape(_BM5, APM, 3 * H).swapaxes(0, 1)


def _node_project(agg, ma, ia, l0, l1, l2, gbias, wif, bif, wir, bir):
    rows = _BM5 * APM
    wspec = pl.BlockSpec((H, H), lambda i: (0, 0))
    w3spec = pl.BlockSpec((H, 3 * H), lambda i: (0, 0))
    b3spec = pl.BlockSpec((1, 3 * H), lambda i: (0, 0))
    return pl.pallas_call(
        _node_body,
        grid=(N_MOL // _BM5,),
        in_specs=[pl.BlockSpec((rows, H), lambda i: (i, 0)),
                  pl.BlockSpec((rows, H), lambda i: (i, 0)),
                  pl.BlockSpec((rows, H), lambda i: (i, 0)),
                  wspec, wspec, wspec,
                  pl.BlockSpec((1, H), lambda i: (0, 0)),
                  w3spec, b3spec, w3spec, b3spec],
        out_specs=[pl.BlockSpec((APM, _BM5, 3 * H), lambda i: (0, i, 0)),
                   pl.BlockSpec((APM, _BM5, 3 * H), lambda i: (0, i, 0)),
                   pl.BlockSpec((_BM5, H), lambda i: (i, 0))],
        out_shape=[jax.ShapeDtypeStruct((APM, N_MOL, 3 * H), jnp.float32),
                   jax.ShapeDtypeStruct((APM, N_MOL, 3 * H), jnp.float32),
                   jax.ShapeDtypeStruct((N_MOL, H), jnp.float32)],
    )(agg, ma, ia, l0, l1, l2, gbias, wif, bif, wir, bir)


def _gru_body(gif_ref, gir_ref, h0_ref, whf_ref, bhf_ref, whr_ref, bhr_ref,
              of_ref, or_ref, hf_s, hr_s):
    t = pl.program_id(0)

    @pl.when(t == 0)
    def _():
        hf_s[...] = h0_ref[...]
        hr_s[...] = h0_ref[...]

    def step(gi, h, wh_ref, bh_ref):
        gh = jnp.dot(h, wh_ref[...], preferred_element_type=jnp.float32) + bh_ref[...]
        r = jax.nn.sigmoid(gi[:, :H] + gh[:, :H])
        z = jax.nn.sigmoid(gi[:, H:2 * H] + gh[:, H:2 * H])
        n = jnp.tanh(gi[:, 2 * H:] + r * gh[:, 2 * H:])
        return (1.0 - z) * n + z * h

    hf = step(gif_ref[...].reshape(N_MOL, 3 * H), hf_s[...], whf_ref, bhf_ref)
    hr = step(gir_ref[...].reshape(N_MOL, 3 * H), hr_s[...], whr_ref, bhr_ref)
    hf_s[...] = hf
    hr_s[...] = hr
    of_ref[...] = hf.reshape(1, N_MOL, H)
    or_ref[...] = hr.reshape(1, N_MOL, H)


def _gru(gif, gir, h0, whf, bhf, whr, bhr):
    w3spec = pl.BlockSpec((H, 3 * H), lambda t: (0, 0))
    b3spec = pl.BlockSpec((1, 3 * H), lambda t: (0, 0))
    return pl.pallas_call(
        _gru_body,
        grid=(APM,),
        in_specs=[pl.BlockSpec((1, N_MOL, 3 * H), lambda t: (t, 0, 0)),
                  pl.BlockSpec((1, N_MOL, 3 * H), lambda t: (APM - 1 - t, 0, 0)),
                  pl.BlockSpec((N_MOL, H), lambda t: (0, 0)),
                  w3spec, b3spec, w3spec, b3spec],
        out_specs=[pl.BlockSpec((1, N_MOL, H), lambda t: (t, 0, 0)),
                   pl.BlockSpec((1, N_MOL, H), lambda t: (APM - 1 - t, 0, 0))],
        out_shape=[jax.ShapeDtypeStruct((APM, N_MOL, H), jnp.float32),
                   jax.ShapeDtypeStruct((APM, N_MOL, H), jnp.float32)],
        scratch_shapes=[pltpu.VMEM((N_MOL, H), jnp.float32),
                        pltpu.VMEM((N_MOL, H), jnp.float32)],
    )(gif, gir, h0, whf, bhf, whr, bhr)


_BM7 = 128


def _readout_body(of_ref, or_ref, wof_ref, wor_ref, b_ref, o_ref):
    f = of_ref[...].reshape(APM * _BM7, H)
    r = or_ref[...].reshape(APM * _BM7, H)
    ah = jax.nn.relu(
        jnp.dot(f, wof_ref[...], preferred_element_type=jnp.float32)
        + jnp.dot(r, wor_ref[...], preferred_element_type=jnp.float32)
        + b_ref[...])
    o_ref[...] = ah.reshape(APM, _BM7, H).sum(axis=0) * (1.0 / APM)


def _readout(of, orr, wof, wor, b):
    wspec = pl.BlockSpec((H, H), lambda i: (0, 0))
    return pl.pallas_call(
        _readout_body,
        grid=(N_MOL // _BM7,),
        in_specs=[pl.BlockSpec((APM, _BM7, H), lambda i: (0, i, 0)),
                  pl.BlockSpec((APM, _BM7, H), lambda i: (0, i, 0)),
                  wspec, wspec,
                  pl.BlockSpec((1, H), lambda i: (0, 0))],
        out_specs=pl.BlockSpec((_BM7, H), lambda i: (i, 0)),
        out_shape=jax.ShapeDtypeStruct((N_MOL, H), jnp.float32),
    )(of, orr, wof, wor, b)


def kernel(f_atoms, f_bonds, a2b, b2a, b2revb, a_scope, W_i_atom, W_i_bond,
           W_h_0, W_h_1, lr_W, W_o_W, W_o_b, gru_bias, W_ih_f, W_hh_f,
           b_ih_f, b_hh_f, W_ih_r, W_hh_r, b_ih_r, b_hh_r):
    del a_scope
    f32 = jnp.float32

    ak = 136
    bk = 152
    fa = jnp.pad(f_atoms, ((0, A_PAD - N_ATOMS), (0, ak - 133)))
    fb = jnp.pad(f_bonds, ((0, B_PAD - N_BONDS), (0, bk - 147)))
    wia_t = jnp.pad(W_i_atom.T, ((0, ak - 133), (0, 0)))
    wib_t = jnp.pad(W_i_bond.T, ((0, bk - 147), (0, 0)))

    ia = _mm_relu(fa, wia_t, 512)
    ib = _mm_relu(fb, wib_t, 512)

    a2b_flat = jnp.pad(a2b.astype(jnp.int32),
                       ((0, A_PAD - N_ATOMS), (0, 0))).reshape(-1)
    b2a_p = jnp.pad(b2a.astype(jnp.int32), (0, B_PAD - N_BONDS))
    b2revb_p = jnp.pad(b2revb.astype(jnp.int32), (0, B_PAD - N_BONDS))

    ma = ia
    mb = ib
    for wh in (W_h_0, W_h_1):
        nei3 = _sc_gather(mb, a2b_flat, 48).reshape(A_PAD, MAX_NB, H)
        ma = _agg_base(nei3, ma)
        ga = _sc_gather(ma, b2a_p, 64)
        gb = _sc_gather(mb, b2revb_p, 64)
        mb = _bond_update(ga, gb, ib, wh.T.astype(f32))

    nei3 = _sc_gather(mb, a2b_flat, 48).reshape(A_PAD, MAX_NB, H)
    agg = _agg_nb(nei3)

    rows = N_MOL * APM
    agg_s = lax.dynamic_slice(agg, (1, 0), (rows, H))
    ma_s = lax.dynamic_slice(ma, (1, 0), (rows, H))
    ia_s = lax.dynamic_slice(ia, (1, 0), (rows, H))

    l0 = lr_W[:, :H].T
    l1 = lr_W[:, H:2 * H].T
    l2 = lr_W[:, 2 * H:].T
    gif, gir, h0 = _node_project(
        agg_s, ma_s, ia_s, l0, l1, l2, gru_bias.reshape(1, H),
        W_ih_f.T, b_ih_f.reshape(1, 3 * H), W_ih_r.T, b_ih_r.reshape(1, 3 * H))

    of, orr = _gru(gif, gir, h0, W_hh_f.T, b_hh_f.reshape(1, 3 * H),
                   W_hh_r.T, b_hh_r.reshape(1, 3 * H))

    return _readout(of, orr, W_o_W[:, :H].T, W_o_W[:, H:].T,
                    W_o_b.reshape(1, H))
